# baseline (speedup 1.0000x reference)
import functools

import jax
import jax.numpy as jnp
from jax import lax
from jax.experimental import pallas as pl
from jax.experimental.pallas import tpu as pltpu
from jax.experimental.pallas import tpu_sc as plsc

_E = 64
_D_IN = 1024
_D_OUT = 1024
_N = 4096
_BM = 128
_NBLK = _N // _BM + _E
_PADN = _NBLK * _BM

_NC = 2
_NS = 16
_NW = _NC * _NS
_CHUNK = _N // _NW
_SUB = 32

_mesh = plsc.VectorSubcoreMesh(core_axis_name="c", subcore_axis_name="s")


_RCH = _N // _NS
_NV = _RCH // 16


def _rank_tot(v, vv_v, iota):
    zero = jnp.zeros((16,), jnp.int32)
    vv_v[pl.ds(0, 16)] = v
    vv_v[pl.ds(16, 16)] = v

    def rot_body(r, carry):
        rank, tot = carry
        prev = vv_v[pl.ds(16 - r, 16)]
        eq = (prev == v).astype(jnp.int32)
        return rank + jnp.where(iota >= r, eq, zero), tot + eq

    rank, tot = jax.lax.fori_loop(1, 16, rot_body, (zero, zero))
    return rank, tot + 1


@functools.partial(
    pl.kernel,
    mesh=_mesh,
    compiler_params=pltpu.CompilerParams(needs_layout_passes=False),
    out_type=[
        jax.ShapeDtypeStruct((_N,), jnp.int32),
        jax.ShapeDtypeStruct((3 * _NBLK + 16,), jnp.int32),
    ],
    scratch_types=[
        pltpu.VMEM((_RCH,), jnp.int32),
        pltpu.VMEM((_RCH,), jnp.int32),
        pltpu.VMEM((_RCH,), jnp.int32),
        pltpu.VMEM((_RCH,), jnp.int32),
        pltpu.VMEM((_E,), jnp.int32),
        pltpu.VMEM((_NS * _E,), jnp.int32),
        pltpu.VMEM((_E,), jnp.int32),
        pltpu.VMEM((3 * _NBLK + 16,), jnp.int32),
        pltpu.VMEM((32,), jnp.int32),
        pltpu.VMEM((_E // 16 * 32,), jnp.int32),
        pltpu.VMEM_SHARED((_NS * _E,), jnp.int32),
    ],
)
def _sc_route(ids_hbm, ppos_hbm, be_hbm, ids_v, rank_v, tot_v, ppos_v,
              hist_v, histall_v, base_v, be_v, vv_v, psd_v, hist_sh):
    cid = lax.axis_index("c")
    wid = lax.axis_index("s")
    iota = jax.lax.iota(jnp.int32, 16)
    base_tok = wid * _RCH

    pltpu.sync_copy(ids_hbm.at[pl.ds(base_tok, _RCH)], ids_v)
    for g in range(_E // 16):
        hist_v[pl.ds(g * 16, 16)] = jnp.zeros((16,), jnp.int32)

    for k in range(_NV):
        v = ids_v[pl.ds(k * 16, 16)]
        rank, tot = _rank_tot(v, vv_v, iota)
        rank_v[pl.ds(k * 16, 16)] = rank
        tot_v[pl.ds(k * 16, 16)] = tot
        plsc.addupdate_scatter(hist_v, [v], tot, mask=rank == tot - 1)

    pltpu.sync_copy(hist_v, hist_sh.at[pl.ds(wid * _E, _E)])
    plsc.subcore_barrier()
    pltpu.sync_copy(hist_sh, histall_v)

    cnts = []
    parts = []
    for g in range(_E // 16):
        def w_body(w, carry, g=g):
            c, p = carry
            h = histall_v[pl.ds(w * _E + g * 16, 16)]
            return c + h, p + jnp.where(w < wid, h, jnp.zeros((16,), jnp.int32))

        c_g, p_g = jax.lax.fori_loop(
            0, _NS, w_body,
            (jnp.zeros((16,), jnp.int32), jnp.zeros((16,), jnp.int32)))
        cnts.append(c_g)
        parts.append(p_g)

    carry = jnp.int32(0)
    pstarts = []
    for g in range(_E // 16):
        pb = (cnts[g] + _BM - 1) // _BM
        s = plsc.cumsum(pb)
        pstarts.append(s - pb + carry)
        carry = carry + jnp.sum(pb)
    for g in range(_E // 16):
        base_v[pl.ds(g * 16, 16)] = pstarts[g] * _BM + parts[g]

    nreal = carry
    @pl.when((cid == 0) & (wid == 0))
    def _():
        for g in range(_E // 16):
            psd_v[pl.ds(g * 32, 16)] = pstarts[g]
            psd_v[pl.ds(g * 32 + 16, 16)] = pstarts[g]
        for ti in range(_NBLK // 16):
            tvec = iota + ti * 16
            live = tvec < nreal
            tvec2 = jnp.minimum(tvec, nreal - 1)
            bev = jnp.full((16,), -1, jnp.int32)
            for g in range(_E // 16):
                def rot_be(r, acc, g=g, tvec2=tvec2):
                    psr = psd_v[pl.ds(g * 32 + r, 16)]
                    return acc + (psr <= tvec2).astype(jnp.int32)

                bev = jax.lax.fori_loop(0, 16, rot_be, bev)
            be_v[pl.ds(ti * 16, 16)] = bev
            be_v[pl.ds(_NBLK + ti * 16, 16)] = jnp.where(
                live, tvec, jnp.zeros((16,), jnp.int32))
            be_v[pl.ds(2 * _NBLK + ti * 16, 16)] = jnp.where(
                live, tvec, jnp.full((16,), _NBLK - 1, jnp.int32))
        be_v[pl.ds(3 * _NBLK, 16)] = jnp.full((16,), 1, jnp.int32) * nreal
        pltpu.sync_copy(be_v, be_hbm)

    for k in range(_NV):
        v = ids_v[pl.ds(k * 16, 16)]
        rank = rank_v[pl.ds(k * 16, 16)]
        tot = tot_v[pl.ds(k * 16, 16)]
        b = plsc.load_gather(base_v, [v])
        pos = b + rank
        ppos_v[pl.ds(k * 16, 16)] = pos
        plsc.store_scatter(base_v, [v], pos + (tot - rank), mask=rank == tot - 1)

    @pl.when(cid == 0)
    def _():
        pltpu.sync_copy(ppos_v, ppos_hbm.at[pl.ds(base_tok, _RCH)])


@functools.partial(
    pl.kernel,
    mesh=_mesh,
    out_type=jax.ShapeDtypeStruct((_PADN, _D_IN), jnp.float32),
    scratch_types=[
        pltpu.VMEM((_SUB,), jnp.int32),
        pltpu.VMEM((_SUB,), jnp.int32),
        pltpu.VMEM((_SUB, _D_IN), jnp.float32),
        pltpu.VMEM((_SUB, _D_IN), jnp.float32),
        pltpu.SemaphoreType.DMA,
        pltpu.SemaphoreType.DMA,
    ],
)
def _sc_dispatch(x_hbm, ppos_hbm, xpad_hbm, idx0, idx1, rows0, rows1,
                 sem0, sem1):
    wid = lax.axis_index("s") * _NC + lax.axis_index("c")
    base = wid * _CHUNK
    bufs = ((idx0, rows0, sem0), (idx1, rows1, sem1))
    pending = [None, None]
    for c in range(_CHUNK // _SUB):
        b = c % 2
        idx_v, rows_v, sem = bufs[b]
        if pending[b] is not None:
            pending[b].wait()
        off = base + c * _SUB
        pltpu.sync_copy(ppos_hbm.at[pl.ds(off, _SUB)], idx_v)
        pltpu.sync_copy(x_hbm.at[pl.ds(off, _SUB)], rows_v)
        pending[b] = pltpu.async_copy(rows_v, xpad_hbm.at[idx_v], sem)
    for p in pending:
        p.wait()


@functools.partial(
    pl.kernel,
    mesh=_mesh,
    out_type=jax.ShapeDtypeStruct((_N, _D_OUT), jnp.float32),
    scratch_types=[
        pltpu.VMEM((_SUB,), jnp.int32),
        pltpu.VMEM((_SUB,), jnp.int32),
        pltpu.VMEM((_SUB, _D_OUT), jnp.float32),
        pltpu.VMEM((_SUB, _D_OUT), jnp.float32),
        pltpu.SemaphoreType.DMA,
        pltpu.SemaphoreType.DMA,
    ],
)
def _sc_combine(ypad_hbm, ppos_hbm, out_hbm, idx0, idx1, rows0, rows1,
                sem0, sem1):
    wid = lax.axis_index("s") * _NC + lax.axis_index("c")
    base = wid * _CHUNK
    bufs = ((idx0, rows0, sem0), (idx1, rows1, sem1))
    pending = [None, None]
    offs = [0, 0]
    for c in range(_CHUNK // _SUB):
        b = c % 2
        idx_v, rows_v, sem = bufs[b]
        if pending[b] is not None:
            pending[b].wait()
            pltpu.sync_copy(rows_v, out_hbm.at[pl.ds(offs[b], _SUB)])
        off = base + c * _SUB
        pltpu.sync_copy(ppos_hbm.at[pl.ds(off, _SUB)], idx_v)
        pending[b] = pltpu.async_copy(ypad_hbm.at[idx_v], rows_v, sem)
        offs[b] = off
    for b in (0, 1):
        idx_v, rows_v, sem = bufs[b]
        pending[b].wait()
        pltpu.sync_copy(rows_v, out_hbm.at[pl.ds(offs[b], _SUB)])


_WSPLIT = 4


def _gmm_body(be_ref, x_ref, *rest):
    w_refs = rest[:_WSPLIT]
    b_ref = rest[_WSPLIT]
    o_ref = rest[_WSPLIT + 1]

    @pl.when(pl.program_id(0) < be_ref[3 * _NBLK])
    def _():
        xb = x_ref[...].astype(jnp.bfloat16)
        brow = b_ref[0, 0].astype(jnp.bfloat16)
        piece = _D_OUT // _WSPLIT
        for i, w_ref in enumerate(w_refs):
            wb = w_ref[0, 0].astype(jnp.bfloat16)
            acc = jax.lax.dot_general(
                xb, wb,
                dimension_numbers=(((1,), (1,)), ((), ())),
                preferred_element_type=jnp.float32,
            )
            y = acc.astype(jnp.bfloat16) + brow[i * piece:(i + 1) * piece]
            o_ref[:, i * piece:(i + 1) * piece] = y.astype(jnp.float32)


def _grouped_matmul(block_expert, x_pad, weight, bias):
    w4 = weight.reshape(_E, _WSPLIT, _D_OUT // _WSPLIT, _D_IN)

    def wspec(i):
        return pl.BlockSpec((1, 1, _D_OUT // _WSPLIT, _D_IN),
                            lambda t, m, i=i: (m[t], i, 0, 0))

    grid_spec = pltpu.PrefetchScalarGridSpec(
        num_scalar_prefetch=1,
        grid=(_NBLK,),
        in_specs=[
            pl.BlockSpec((_BM, _D_IN), lambda t, m: (m[_NBLK + t], 0)),
            *[wspec(i) for i in range(_WSPLIT)],
            pl.BlockSpec((1, 1, _D_OUT), lambda t, m: (m[t], 0, 0)),
        ],
        out_specs=pl.BlockSpec((_BM, _D_OUT), lambda t, m: (m[2 * _NBLK + t], 0)),
    )
    return pl.pallas_call(
        _gmm_body,
        grid_spec=grid_spec,
        out_shape=jax.ShapeDtypeStruct((_PADN, _D_OUT), jnp.float32),
    )(block_expert, x_pad, *([w4] * _WSPLIT), bias.reshape(_E, 1, _D_OUT))


def kernel(x, ids, weight, bias):
    ids = ids.reshape(-1).astype(jnp.int32)

    ppos, block_expert = _sc_route(ids)
    x_pad = _sc_dispatch(x.reshape(-1, _D_IN), ppos)
    y_pad = _grouped_matmul(block_expert, x_pad, weight, bias)
    out = _sc_combine(y_pad, ppos)
    return out.reshape(x.shape[:-1] + (_D_OUT,))


# speedup vs baseline: 1.0337x; 1.0337x over previous
import functools

import jax
import jax.numpy as jnp
from jax import lax
from jax.experimental import pallas as pl
from jax.experimental.pallas import tpu as pltpu
from jax.experimental.pallas import tpu_sc as plsc

_E = 64
_D_IN = 1024
_D_OUT = 1024
_N = 4096
_BM = 128
_NBLK = _N // _BM + _E
_PADN = _NBLK * _BM

_NC = 2
_NS = 16
_NW = _NC * _NS
_CHUNK = _N // _NW
_SUB = 32

_mesh = plsc.VectorSubcoreMesh(core_axis_name="c", subcore_axis_name="s")


_RCH = _N // _NS
_NV = _RCH // 16


def _rank_tot(v, vv_v, iota):
    zero = jnp.zeros((16,), jnp.int32)
    vv_v[pl.ds(0, 16)] = v
    vv_v[pl.ds(16, 16)] = v

    def rot_body(r, carry):
        rank, tot = carry
        prev = vv_v[pl.ds(16 - r, 16)]
        eq = (prev == v).astype(jnp.int32)
        return rank + jnp.where(iota >= r, eq, zero), tot + eq

    rank, tot = jax.lax.fori_loop(1, 16, rot_body, (zero, zero))
    return rank, tot + 1


@functools.partial(
    pl.kernel,
    mesh=_mesh,
    compiler_params=pltpu.CompilerParams(needs_layout_passes=False),
    out_type=[
        jax.ShapeDtypeStruct((_N,), jnp.int32),
        jax.ShapeDtypeStruct((3 * _NBLK + 16,), jnp.int32),
        jax.ShapeDtypeStruct((_PADN, _D_IN), jnp.float32),
    ],
    scratch_types=[
        pltpu.VMEM((_RCH,), jnp.int32),
        pltpu.VMEM((_RCH,), jnp.int32),
        pltpu.VMEM((_RCH,), jnp.int32),
        pltpu.VMEM((_RCH,), jnp.int32),
        pltpu.VMEM((_E,), jnp.int32),
        pltpu.VMEM((_NS * _E,), jnp.int32),
        pltpu.VMEM((_E,), jnp.int32),
        pltpu.VMEM((3 * _NBLK + 16,), jnp.int32),
        pltpu.VMEM((32,), jnp.int32),
        pltpu.VMEM((_E // 16 * 32,), jnp.int32),
        pltpu.VMEM_SHARED((_NS * _E,), jnp.int32),
        pltpu.VMEM((_SUB,), jnp.int32),
        pltpu.VMEM((_SUB,), jnp.int32),
        pltpu.VMEM((_SUB, _D_IN), jnp.float32),
        pltpu.VMEM((_SUB, _D_IN), jnp.float32),
        pltpu.SemaphoreType.DMA,
        pltpu.SemaphoreType.DMA,
    ],
)
def _sc_route(ids_hbm, x_hbm, ppos_hbm, be_hbm, xpad_hbm, ids_v, rank_v,
              tot_v, ppos_v, hist_v, histall_v, base_v, be_v, vv_v, psd_v,
              hist_sh, idx0, idx1, rows0, rows1, sem0, sem1):
    cid = lax.axis_index("c")
    wid = lax.axis_index("s")
    iota = jax.lax.iota(jnp.int32, 16)
    base_tok = wid * _RCH

    pltpu.sync_copy(ids_hbm.at[pl.ds(base_tok, _RCH)], ids_v)
    for g in range(_E // 16):
        hist_v[pl.ds(g * 16, 16)] = jnp.zeros((16,), jnp.int32)

    for k in range(_NV):
        v = ids_v[pl.ds(k * 16, 16)]
        rank, tot = _rank_tot(v, vv_v, iota)
        rank_v[pl.ds(k * 16, 16)] = rank
        tot_v[pl.ds(k * 16, 16)] = tot
        plsc.addupdate_scatter(hist_v, [v], tot, mask=rank == tot - 1)

    pltpu.sync_copy(hist_v, hist_sh.at[pl.ds(wid * _E, _E)])
    plsc.subcore_barrier()
    pltpu.sync_copy(hist_sh, histall_v)

    cnts = []
    parts = []
    for g in range(_E // 16):
        def w_body(w, carry, g=g):
            c, p = carry
            h = histall_v[pl.ds(w * _E + g * 16, 16)]
            return c + h, p + jnp.where(w < wid, h, jnp.zeros((16,), jnp.int32))

        c_g, p_g = jax.lax.fori_loop(
            0, _NS, w_body,
            (jnp.zeros((16,), jnp.int32), jnp.zeros((16,), jnp.int32)))
        cnts.append(c_g)
        parts.append(p_g)

    carry = jnp.int32(0)
    pstarts = []
    for g in range(_E // 16):
        pb = (cnts[g] + _BM - 1) // _BM
        s = plsc.cumsum(pb)
        pstarts.append(s - pb + carry)
        carry = carry + jnp.sum(pb)
    for g in range(_E // 16):
        base_v[pl.ds(g * 16, 16)] = pstarts[g] * _BM + parts[g]

    nreal = carry
    @pl.when((cid == 0) & (wid == 0))
    def _():
        for g in range(_E // 16):
            psd_v[pl.ds(g * 32, 16)] = pstarts[g]
            psd_v[pl.ds(g * 32 + 16, 16)] = pstarts[g]
        for ti in range(_NBLK // 16):
            tvec = iota + ti * 16
            live = tvec < nreal
            tvec2 = jnp.minimum(tvec, nreal - 1)
            bev = jnp.full((16,), -1, jnp.int32)
            for g in range(_E // 16):
                def rot_be(r, acc, g=g, tvec2=tvec2):
                    psr = psd_v[pl.ds(g * 32 + r, 16)]
                    return acc + (psr <= tvec2).astype(jnp.int32)

                bev = jax.lax.fori_loop(0, 16, rot_be, bev)
            be_v[pl.ds(ti * 16, 16)] = bev
            be_v[pl.ds(_NBLK + ti * 16, 16)] = jnp.where(
                live, tvec, jnp.zeros((16,), jnp.int32))
            be_v[pl.ds(2 * _NBLK + ti * 16, 16)] = jnp.where(
                live, tvec, jnp.full((16,), _NBLK - 1, jnp.int32))
        be_v[pl.ds(3 * _NBLK, 16)] = jnp.full((16,), 1, jnp.int32) * nreal
        pltpu.sync_copy(be_v, be_hbm)

    for k in range(_NV):
        v = ids_v[pl.ds(k * 16, 16)]
        rank = rank_v[pl.ds(k * 16, 16)]
        tot = tot_v[pl.ds(k * 16, 16)]
        b = plsc.load_gather(base_v, [v])
        pos = b + rank
        ppos_v[pl.ds(k * 16, 16)] = pos
        plsc.store_scatter(base_v, [v], pos + (tot - rank), mask=rank == tot - 1)

    @pl.when(cid == 0)
    def _():
        pltpu.sync_copy(ppos_v, ppos_hbm.at[pl.ds(base_tok, _RCH)])

    half = _RCH // _NC
    tok0 = base_tok + cid * half
    loc0 = cid * half
    bufs = ((idx0, rows0, sem0), (idx1, rows1, sem1))
    pending = [None, None]
    for c in range(half // _SUB):
        b = c % 2
        idx_v, rows_v, sem = bufs[b]
        if pending[b] is not None:
            pending[b].wait()
        for h in range(_SUB // 16):
            idx_v[pl.ds(h * 16, 16)] = ppos_v[pl.ds(loc0 + c * _SUB + h * 16, 16)]
        pltpu.sync_copy(x_hbm.at[pl.ds(tok0 + c * _SUB, _SUB)], rows_v)
        pending[b] = pltpu.async_copy(rows_v, xpad_hbm.at[idx_v], sem)
    for p in pending:
        p.wait()


@functools.partial(
    pl.kernel,
    mesh=_mesh,
    out_type=jax.ShapeDtypeStruct((_N, _D_OUT), jnp.float32),
    scratch_types=[
        pltpu.VMEM((_SUB,), jnp.int32),
        pltpu.VMEM((_SUB,), jnp.int32),
        pltpu.VMEM((_SUB, _D_OUT), jnp.float32),
        pltpu.VMEM((_SUB, _D_OUT), jnp.float32),
        pltpu.SemaphoreType.DMA,
        pltpu.SemaphoreType.DMA,
    ],
)
def _sc_combine(ypad_hbm, ppos_hbm, out_hbm, idx0, idx1, rows0, rows1,
                sem0, sem1):
    wid = lax.axis_index("s") * _NC + lax.axis_index("c")
    base = wid * _CHUNK
    bufs = ((idx0, rows0, sem0), (idx1, rows1, sem1))
    pending = [None, None]
    offs = [0, 0]
    for c in range(_CHUNK // _SUB):
        b = c % 2
        idx_v, rows_v, sem = bufs[b]
        if pending[b] is not None:
            pending[b].wait()
            pltpu.sync_copy(rows_v, out_hbm.at[pl.ds(offs[b], _SUB)])
        off = base + c * _SUB
        pltpu.sync_copy(ppos_hbm.at[pl.ds(off, _SUB)], idx_v)
        pending[b] = pltpu.async_copy(ypad_hbm.at[idx_v], rows_v, sem)
        offs[b] = off
    for b in (0, 1):
        idx_v, rows_v, sem = bufs[b]
        pending[b].wait()
        pltpu.sync_copy(rows_v, out_hbm.at[pl.ds(offs[b], _SUB)])


_WSPLIT = 2


def _gmm_body(be_ref, x_ref, *rest):
    w_refs = rest[:_WSPLIT]
    b_ref = rest[_WSPLIT]
    o_ref = rest[_WSPLIT + 1]

    @pl.when(pl.program_id(0) < be_ref[3 * _NBLK])
    def _():
        xb = x_ref[...].astype(jnp.bfloat16)
        brow = b_ref[0, 0].astype(jnp.bfloat16)
        piece = _D_OUT // _WSPLIT
        for i, w_ref in enumerate(w_refs):
            wb = w_ref[0, 0].astype(jnp.bfloat16)
            acc = jax.lax.dot_general(
                xb, wb,
                dimension_numbers=(((1,), (1,)), ((), ())),
                preferred_element_type=jnp.float32,
            )
            y = acc.astype(jnp.bfloat16) + brow[i * piece:(i + 1) * piece]
            o_ref[:, i * piece:(i + 1) * piece] = y.astype(jnp.float32)


def _grouped_matmul(block_expert, x_pad, weight, bias):
    w4 = weight.reshape(_E, _WSPLIT, _D_OUT // _WSPLIT, _D_IN)

    def wspec(i):
        return pl.BlockSpec((1, 1, _D_OUT // _WSPLIT, _D_IN),
                            lambda t, m, i=i: (m[t], i, 0, 0))

    grid_spec = pltpu.PrefetchScalarGridSpec(
        num_scalar_prefetch=1,
        grid=(_NBLK,),
        in_specs=[
            pl.BlockSpec((_BM, _D_IN), lambda t, m: (m[_NBLK + t], 0)),
            *[wspec(i) for i in range(_WSPLIT)],
            pl.BlockSpec((1, 1, _D_OUT), lambda t, m: (m[t], 0, 0)),
        ],
        out_specs=pl.BlockSpec((_BM, _D_OUT), lambda t, m: (m[2 * _NBLK + t], 0)),
    )
    return pl.pallas_call(
        _gmm_body,
        grid_spec=grid_spec,
        out_shape=jax.ShapeDtypeStruct((_PADN, _D_OUT), jnp.float32),
    )(block_expert, x_pad, *([w4] * _WSPLIT), bias.reshape(_E, 1, _D_OUT))


def kernel(x, ids, weight, bias):
    ids = ids.reshape(-1).astype(jnp.int32)

    ppos, block_expert, x_pad = _sc_route(ids, x.reshape(-1, _D_IN))
    y_pad = _grouped_matmul(block_expert, x_pad, weight, bias)
    out = _sc_combine(y_pad, ppos)
    return out.reshape(x.shape[:-1] + (_D_OUT,))


# speedup vs baseline: 1.0371x; 1.0032x over previous
import functools

import jax
import jax.numpy as jnp
from jax import lax
from jax.experimental import pallas as pl
from jax.experimental.pallas import tpu as pltpu
from jax.experimental.pallas import tpu_sc as plsc

_E = 64
_D_IN = 1024
_D_OUT = 1024
_N = 4096
_BM = 128
_NBLK = _N // _BM + _E
_PADN = _NBLK * _BM

_NC = 2
_NS = 16
_NW = _NC * _NS
_CHUNK = _N // _NW
_SUB = 32

_mesh = plsc.VectorSubcoreMesh(core_axis_name="c", subcore_axis_name="s")


_RCH = _N // _NS
_NV = _RCH // 16


def _rank_tot(v, vv_v, iota):
    zero = jnp.zeros((16,), jnp.int32)
    vv_v[pl.ds(0, 16)] = v
    vv_v[pl.ds(16, 16)] = v

    def rot_body(r, carry):
        rank, tot = carry
        prev = vv_v[pl.ds(16 - r, 16)]
        eq = (prev == v).astype(jnp.int32)
        return rank + jnp.where(iota >= r, eq, zero), tot + eq

    rank, tot = jax.lax.fori_loop(1, 16, rot_body, (zero, zero))
    return rank, tot + 1


@functools.partial(
    pl.kernel,
    mesh=_mesh,
    compiler_params=pltpu.CompilerParams(needs_layout_passes=False),
    out_type=[
        jax.ShapeDtypeStruct((_N,), jnp.int32),
        jax.ShapeDtypeStruct((3 * _NBLK + 16,), jnp.int32),
        jax.ShapeDtypeStruct((_PADN, _D_IN), jnp.float32),
    ],
    scratch_types=[
        pltpu.VMEM((_RCH,), jnp.int32),
        pltpu.VMEM((_RCH,), jnp.int32),
        pltpu.VMEM((_RCH,), jnp.int32),
        pltpu.VMEM((_RCH,), jnp.int32),
        pltpu.VMEM((_E,), jnp.int32),
        pltpu.VMEM((_NS * _E,), jnp.int32),
        pltpu.VMEM((_E,), jnp.int32),
        pltpu.VMEM((3 * _NBLK + 16,), jnp.int32),
        pltpu.VMEM((32,), jnp.int32),
        pltpu.VMEM((_E // 16 * 32,), jnp.int32),
        pltpu.VMEM_SHARED((_NS * _E,), jnp.int32),
        pltpu.VMEM((_SUB,), jnp.int32),
        pltpu.VMEM((_SUB,), jnp.int32),
        pltpu.VMEM((_SUB, _D_IN), jnp.float32),
        pltpu.VMEM((_SUB, _D_IN), jnp.float32),
        pltpu.SemaphoreType.DMA,
        pltpu.SemaphoreType.DMA,
    ],
)
def _sc_route(ids_hbm, x_hbm, ppos_hbm, be_hbm, xpad_hbm, ids_v, rank_v,
              tot_v, ppos_v, hist_v, histall_v, base_v, be_v, vv_v, psd_v,
              hist_sh, idx0, idx1, rows0, rows1, sem0, sem1):
    cid = lax.axis_index("c")
    wid = lax.axis_index("s")
    iota = jax.lax.iota(jnp.int32, 16)
    base_tok = wid * _RCH

    pltpu.sync_copy(ids_hbm.at[pl.ds(base_tok, _RCH)], ids_v)
    for g in range(_E // 16):
        hist_v[pl.ds(g * 16, 16)] = jnp.zeros((16,), jnp.int32)

    for k in range(_NV):
        v = ids_v[pl.ds(k * 16, 16)]
        rank, tot = _rank_tot(v, vv_v, iota)
        rank_v[pl.ds(k * 16, 16)] = rank
        tot_v[pl.ds(k * 16, 16)] = tot
        plsc.addupdate_scatter(hist_v, [v], tot, mask=rank == tot - 1)

    pltpu.sync_copy(hist_v, hist_sh.at[pl.ds(wid * _E, _E)])
    plsc.subcore_barrier()
    pltpu.sync_copy(hist_sh, histall_v)

    cnts = []
    parts = []
    for g in range(_E // 16):
        def w_body(w, carry, g=g):
            c, p = carry
            h = histall_v[pl.ds(w * _E + g * 16, 16)]
            return c + h, p + jnp.where(w < wid, h, jnp.zeros((16,), jnp.int32))

        c_g, p_g = jax.lax.fori_loop(
            0, _NS, w_body,
            (jnp.zeros((16,), jnp.int32), jnp.zeros((16,), jnp.int32)))
        cnts.append(c_g)
        parts.append(p_g)

    carry = jnp.int32(0)
    pstarts = []
    for g in range(_E // 16):
        pb = (cnts[g] + _BM - 1) // _BM
        s = plsc.cumsum(pb)
        pstarts.append(s - pb + carry)
        carry = carry + jnp.sum(pb)
    for g in range(_E // 16):
        base_v[pl.ds(g * 16, 16)] = pstarts[g] * _BM + parts[g]

    nreal = carry
    @pl.when((cid == 0) & (wid == 0))
    def _():
        for g in range(_E // 16):
            psd_v[pl.ds(g * 32, 16)] = pstarts[g]
            psd_v[pl.ds(g * 32 + 16, 16)] = pstarts[g]
        for ti in range(_NBLK // 16):
            tvec = iota + ti * 16
            live = tvec < nreal
            tvec2 = jnp.minimum(tvec, nreal - 1)
            bev = jnp.full((16,), -1, jnp.int32)
            for g in range(_E // 16):
                def rot_be(r, acc, g=g, tvec2=tvec2):
                    psr = psd_v[pl.ds(g * 32 + r, 16)]
                    return acc + (psr <= tvec2).astype(jnp.int32)

                bev = jax.lax.fori_loop(0, 16, rot_be, bev)
            be_v[pl.ds(ti * 16, 16)] = bev
            be_v[pl.ds(_NBLK + ti * 16, 16)] = jnp.where(
                live, tvec, jnp.zeros((16,), jnp.int32))
            be_v[pl.ds(2 * _NBLK + ti * 16, 16)] = jnp.where(
                live, tvec, jnp.full((16,), _NBLK - 1, jnp.int32))
        be_v[pl.ds(3 * _NBLK, 16)] = jnp.full((16,), 1, jnp.int32) * nreal
        pltpu.sync_copy(be_v, be_hbm)

    for k in range(_NV):
        v = ids_v[pl.ds(k * 16, 16)]
        rank = rank_v[pl.ds(k * 16, 16)]
        tot = tot_v[pl.ds(k * 16, 16)]
        b = plsc.load_gather(base_v, [v])
        pos = b + rank
        ppos_v[pl.ds(k * 16, 16)] = pos
        plsc.store_scatter(base_v, [v], pos + (tot - rank), mask=rank == tot - 1)

    @pl.when(cid == 0)
    def _():
        pltpu.sync_copy(ppos_v, ppos_hbm.at[pl.ds(base_tok, _RCH)])

    half = _RCH // _NC
    tok0 = base_tok + cid * half
    loc0 = cid * half
    bufs = ((idx0, rows0, sem0), (idx1, rows1, sem1))
    pending = [None, None]
    for c in range(half // _SUB):
        b = c % 2
        idx_v, rows_v, sem = bufs[b]
        if pending[b] is not None:
            pending[b].wait()
        for h in range(_SUB // 16):
            idx_v[pl.ds(h * 16, 16)] = ppos_v[pl.ds(loc0 + c * _SUB + h * 16, 16)]
        pltpu.sync_copy(x_hbm.at[pl.ds(tok0 + c * _SUB, _SUB)], rows_v)
        pending[b] = pltpu.async_copy(rows_v, xpad_hbm.at[idx_v], sem)
    for p in pending:
        p.wait()


@functools.partial(
    pl.kernel,
    mesh=_mesh,
    out_type=jax.ShapeDtypeStruct((_N, _D_OUT), jnp.float32),
    scratch_types=[
        pltpu.VMEM((_SUB,), jnp.int32),
        pltpu.VMEM((_SUB,), jnp.int32),
        pltpu.VMEM((_SUB, _D_OUT), jnp.float32),
        pltpu.VMEM((_SUB, _D_OUT), jnp.float32),
        pltpu.SemaphoreType.DMA,
        pltpu.SemaphoreType.DMA,
    ],
)
def _sc_combine(ypad_hbm, ppos_hbm, out_hbm, idx0, idx1, rows0, rows1,
                sem0, sem1):
    wid = lax.axis_index("s") * _NC + lax.axis_index("c")
    base = wid * _CHUNK
    bufs = ((idx0, rows0, sem0), (idx1, rows1, sem1))
    pending = [None, None]
    offs = [0, 0]
    for c in range(_CHUNK // _SUB):
        b = c % 2
        idx_v, rows_v, sem = bufs[b]
        if pending[b] is not None:
            pending[b].wait()
            pltpu.sync_copy(rows_v, out_hbm.at[pl.ds(offs[b], _SUB)])
        off = base + c * _SUB
        pltpu.sync_copy(ppos_hbm.at[pl.ds(off, _SUB)], idx_v)
        pending[b] = pltpu.async_copy(ypad_hbm.at[idx_v], rows_v, sem)
        offs[b] = off
    for b in (0, 1):
        idx_v, rows_v, sem = bufs[b]
        pending[b].wait()
        pltpu.sync_copy(rows_v, out_hbm.at[pl.ds(offs[b], _SUB)])


_WSPLIT = 1


def _gmm_body(be_ref, x_ref, *rest):
    w_refs = rest[:_WSPLIT]
    b_ref = rest[_WSPLIT]
    o_ref = rest[_WSPLIT + 1]

    @pl.when(pl.program_id(0) < be_ref[3 * _NBLK])
    def _():
        xb = x_ref[...].astype(jnp.bfloat16)
        brow = b_ref[0, 0].astype(jnp.bfloat16)
        piece = _D_OUT // _WSPLIT
        for i, w_ref in enumerate(w_refs):
            wb = w_ref[0, 0].astype(jnp.bfloat16)
            acc = jax.lax.dot_general(
                xb, wb,
                dimension_numbers=(((1,), (1,)), ((), ())),
                preferred_element_type=jnp.float32,
            )
            y = acc.astype(jnp.bfloat16) + brow[i * piece:(i + 1) * piece]
            o_ref[:, i * piece:(i + 1) * piece] = y.astype(jnp.float32)


def _grouped_matmul(block_expert, x_pad, weight, bias):
    w4 = weight.reshape(_E, _WSPLIT, _D_OUT // _WSPLIT, _D_IN)

    def wspec(i):
        return pl.BlockSpec((1, 1, _D_OUT // _WSPLIT, _D_IN),
                            lambda t, m, i=i: (m[t], i, 0, 0))

    grid_spec = pltpu.PrefetchScalarGridSpec(
        num_scalar_prefetch=1,
        grid=(_NBLK,),
        in_specs=[
            pl.BlockSpec((_BM, _D_IN), lambda t, m: (m[_NBLK + t], 0)),
            *[wspec(i) for i in range(_WSPLIT)],
            pl.BlockSpec((1, 1, _D_OUT), lambda t, m: (m[t], 0, 0)),
        ],
        out_specs=pl.BlockSpec((_BM, _D_OUT), lambda t, m: (m[2 * _NBLK + t], 0)),
    )
    return pl.pallas_call(
        _gmm_body,
        grid_spec=grid_spec,
        out_shape=jax.ShapeDtypeStruct((_PADN, _D_OUT), jnp.float32),
    )(block_expert, x_pad, *([w4] * _WSPLIT), bias.reshape(_E, 1, _D_OUT))


def kernel(x, ids, weight, bias):
    ids = ids.reshape(-1).astype(jnp.int32)

    ppos, block_expert, x_pad = _sc_route(ids, x.reshape(-1, _D_IN))
    y_pad = _grouped_matmul(block_expert, x_pad, weight, bias)
    out = _sc_combine(y_pad, ppos)
    return out.reshape(x.shape[:-1] + (_D_OUT,))
